# baseline (speedup 1.0000x reference)
import functools

import jax
import jax.numpy as jnp
from jax import lax
from jax.experimental import pallas as pl
from jax.experimental.pallas import tpu as pltpu
from jax.experimental.pallas import tpu_sc as plsc

_TAU = 1.0
_C = 14
_LOSS_WEIGHT = 1.0

_H_TC = 256
_BLK_TC = 128
_BLK_FIN = 128
_SC_ROWS = 32
_RG = 8
_HALF = 7
_W = 512


def _fold(x):
    h, w = x.shape
    while w > 128:
        w //= 2
        x = x[:, :w] + x[:, w:]
    while h > 8:
        h //= 2
        x = x[:h, :] + x[h:, :]
    return x


def _segment_accumulate(b, gt, kl, acc_s, acc_c):
    ones = jnp.ones_like(kl)
    for c in range(1, _C):
        m = gt == c
        fk = _fold(jnp.where(m, kl, 0.0))
        fc = _fold(jnp.where(m, ones, 0.0))
        row = pl.ds(b * 128 + c * 8, 8)
        acc_s[row, :] += fk
        acc_c[row, :] += fc


def _dense_partial_kernel(gt_ref, s_ref, t_ref, accs_ref, accc_ref, *, n_hblk, n_b):
    b = pl.program_id(0)
    ih = pl.program_id(1)

    @pl.when(jnp.logical_and(b == 0, ih == 0))
    def _init():
        accs_ref[...] = jnp.zeros_like(accs_ref)
        accc_ref[...] = jnp.zeros_like(accc_ref)

    se = te = we = None
    for c in range(_C):
        sc_ = s_ref[0, c]
        tc_ = t_ref[0, c]
        if _TAU != 1.0:
            sc_ = sc_ / _TAU
            tc_ = tc_ / _TAU
        esc = jnp.exp(sc_)
        etc_ = jnp.exp(tc_)
        wc = etc_ * (tc_ - sc_)
        if se is None:
            se, te, we = esc, etc_, wc
        else:
            se = se + esc
            te = te + etc_
            we = we + wc

    kl = we / te + jnp.log(se) - jnp.log(te)
    _segment_accumulate(b, gt_ref[0, 0], kl, accs_ref, accc_ref)


def _finish_kernel(gt_ref, se_ref, te_ref, we_ref, accs_in, accc_in, out_ref,
                   acc_s, acc_c, *, n_hblk, n_b):
    b = pl.program_id(0)
    ih = pl.program_id(1)

    @pl.when(jnp.logical_and(b == 0, ih == 0))
    def _init():
        acc_s[...] = accs_in[...]
        acc_c[...] = accc_in[...]

    se = se_ref[0]
    te = te_ref[0]
    we = we_ref[0]
    kl = we / te + jnp.log(se) - jnp.log(te)
    _segment_accumulate(b, gt_ref[0, 0], kl, acc_s, acc_c)

    @pl.when(jnp.logical_and(b == n_b - 1, ih == n_hblk - 1))
    def _finish():
        sums3 = acc_s[...].reshape(64, 8, 128)
        cnts3 = acc_c[...].reshape(64, 8, 128)
        sums = jnp.sum(jnp.sum(sums3, axis=1), axis=1, keepdims=True)
        cnts = jnp.sum(jnp.sum(cnts3, axis=1), axis=1, keepdims=True)
        rid = jax.lax.broadcasted_iota(jnp.int32, sums.shape, 0)
        cid = jax.lax.bitwise_and(rid, 15)
        valid = jnp.logical_and(cid >= 1, cid <= _C - 1)
        valid = jnp.logical_and(valid, cnts > 0.0)
        per = jnp.where(valid, sums / (_C * jnp.maximum(cnts, 1.0)), 0.0)
        out_ref[...] = jnp.sum(per, axis=0, keepdims=True) * (_TAU ** 2) * _LOSS_WEIGHT


def _sc_sums_body(s_hbm, t_hbm, se_hbm, te_hbm, we_hbm,
                  sbuf0, tbuf0, sbuf1, tbuf1, ase, ate, awe, sem0, sem1, semf):
    wid = lax.axis_index("c") * 16 + lax.axis_index("s")
    b = wid // 8
    hblk = wid % 8
    bufs = ((sbuf0, tbuf0, sem0), (sbuf1, tbuf1, sem1))
    n_vec = _W // 16

    def fire(slot, g, p):
        sb, tb, sem = bufs[slot]
        h0 = _H_TC + hblk * _SC_ROWS + g * _RG
        handles = []
        for i in range(_HALF):
            c = p * _HALF + i
            handles.append(pltpu.async_copy(
                s_hbm.at[b, c, pl.ds(h0, _RG), :], sb.at[i], sem))
            handles.append(pltpu.async_copy(
                t_hbm.at[b, c, pl.ds(h0, _RG), :], tb.at[i], sem))
        return handles

    def compute(slot, p):
        sb, tb, _ = bufs[slot]
        first = p == 0
        unroll = 4

        def row_body(r, _):
            def vec_body(j4, _):
                for u in range(unroll):
                    off = pl.ds((j4 * unroll + u) * 16, 16)
                    if first:
                        se_v = jnp.zeros((16,), jnp.float32)
                        te_v = jnp.zeros((16,), jnp.float32)
                        we_v = jnp.zeros((16,), jnp.float32)
                    else:
                        se_v = ase[r, off]
                        te_v = ate[r, off]
                        we_v = awe[r, off]
                    for i in range(_HALF):
                        sv = sb[i, r, off]
                        tv = tb[i, r, off]
                        if _TAU != 1.0:
                            sv = sv / _TAU
                            tv = tv / _TAU
                        es = jnp.exp(sv)
                        et = jnp.exp(tv)
                        se_v = se_v + es
                        te_v = te_v + et
                        we_v = we_v + et * (tv - sv)
                    ase[r, off] = se_v
                    ate[r, off] = te_v
                    awe[r, off] = we_v
                return 0

            return lax.fori_loop(0, n_vec // unroll, vec_body, 0)

        lax.fori_loop(0, _RG, row_body, 0)

    def flush(g):
        h0o = hblk * _SC_ROWS + g * _RG
        return [
            pltpu.async_copy(ase, se_hbm.at[b, pl.ds(h0o, _RG), :], semf),
            pltpu.async_copy(ate, te_hbm.at[b, pl.ds(h0o, _RG), :], semf),
            pltpu.async_copy(awe, we_hbm.at[b, pl.ds(h0o, _RG), :], semf),
        ]

    seq = [(g, p) for g in range(_SC_ROWS // _RG) for p in range(_C // _HALF)]
    pending = {0: fire(0, *seq[0])}
    out_pending = []
    for idx, (g, p) in enumerate(seq):
        slot = idx % 2
        for h in pending.pop(slot):
            h.wait()
        if idx + 1 < len(seq):
            pending[(idx + 1) % 2] = fire((idx + 1) % 2, *seq[idx + 1])
        if p == 0:
            for h in out_pending:
                h.wait()
            out_pending = []
        compute(slot, p)
        if p == _C // _HALF - 1:
            out_pending = flush(g)
    for h in out_pending:
        h.wait()


def _sc_sums(preds_S, preds_T):
    B = preds_S.shape[0]
    h_sc = preds_S.shape[2] - _H_TC
    shp = jax.ShapeDtypeStruct((B, h_sc, _W), jnp.float32)
    fn = pl.kernel(
        _sc_sums_body,
        out_type=[shp, shp, shp],
        mesh=plsc.VectorSubcoreMesh(core_axis_name="c", subcore_axis_name="s"),
        scratch_types=[
            pltpu.VMEM((_HALF, _RG, _W), jnp.float32),
            pltpu.VMEM((_HALF, _RG, _W), jnp.float32),
            pltpu.VMEM((_HALF, _RG, _W), jnp.float32),
            pltpu.VMEM((_HALF, _RG, _W), jnp.float32),
            pltpu.VMEM((_RG, _W), jnp.float32),
            pltpu.VMEM((_RG, _W), jnp.float32),
            pltpu.VMEM((_RG, _W), jnp.float32),
            pltpu.SemaphoreType.DMA,
            pltpu.SemaphoreType.DMA,
            pltpu.SemaphoreType.DMA,
        ],
    )
    return fn(preds_S, preds_T)


def kernel(preds_S, preds_T, gt_labels):
    B, C, H, W = preds_S.shape
    gt = gt_labels.astype(jnp.int32)
    h_sc = H - _H_TC

    se, te, we = _sc_sums(preds_S, preds_T)

    accs, accc = pl.pallas_call(
        functools.partial(_dense_partial_kernel, n_hblk=_H_TC // _BLK_TC, n_b=B),
        grid=(B, _H_TC // _BLK_TC),
        in_specs=[
            pl.BlockSpec((1, 1, _BLK_TC, W), lambda b, ih: (b, 0, ih, 0)),
            pl.BlockSpec((1, C, _BLK_TC, W), lambda b, ih: (b, 0, ih, 0)),
            pl.BlockSpec((1, C, _BLK_TC, W), lambda b, ih: (b, 0, ih, 0)),
        ],
        out_specs=[
            pl.BlockSpec((512, 128), lambda b, ih: (0, 0)),
            pl.BlockSpec((512, 128), lambda b, ih: (0, 0)),
        ],
        out_shape=[
            jax.ShapeDtypeStruct((512, 128), jnp.float32),
            jax.ShapeDtypeStruct((512, 128), jnp.float32),
        ],
        compiler_params=pltpu.CompilerParams(
            dimension_semantics=("arbitrary", "arbitrary"),
        ),
    )(gt, preds_S, preds_T)

    n_fin = h_sc // _BLK_FIN
    off = _H_TC // _BLK_FIN
    out = pl.pallas_call(
        functools.partial(_finish_kernel, n_hblk=n_fin, n_b=B),
        grid=(B, n_fin),
        in_specs=[
            pl.BlockSpec((1, 1, _BLK_FIN, W), lambda b, ih: (b, 0, ih + off, 0)),
            pl.BlockSpec((1, _BLK_FIN, W), lambda b, ih: (b, ih, 0)),
            pl.BlockSpec((1, _BLK_FIN, W), lambda b, ih: (b, ih, 0)),
            pl.BlockSpec((1, _BLK_FIN, W), lambda b, ih: (b, ih, 0)),
            pl.BlockSpec((512, 128), lambda b, ih: (0, 0)),
            pl.BlockSpec((512, 128), lambda b, ih: (0, 0)),
        ],
        out_specs=pl.BlockSpec((1, 1), lambda b, ih: (0, 0)),
        out_shape=jax.ShapeDtypeStruct((1, 1), jnp.float32),
        scratch_shapes=[
            pltpu.VMEM((512, 128), jnp.float32),
            pltpu.VMEM((512, 128), jnp.float32),
        ],
        compiler_params=pltpu.CompilerParams(
            dimension_semantics=("arbitrary", "arbitrary"),
        ),
    )(gt, se, te, we, accs, accc)
    return out[0, 0]


# speedup vs baseline: 1.6436x; 1.6436x over previous
import functools

import jax
import jax.numpy as jnp
from jax.experimental import pallas as pl
from jax.experimental.pallas import tpu as pltpu

_TAU = 1.0
_C = 14
_LOSS_WEIGHT = 1.0


def _fold(x):
    h, w = x.shape
    while w > 128:
        w //= 2
        x = x[:, :w] + x[:, w:]
    while h > 8:
        h //= 2
        x = x[:h, :] + x[h:, :]
    return x


def _kl_loss_kernel(gt_ref, s_ref, t_ref, out_ref, acc_s, acc_c, *, n_hblk, n_b):
    b = pl.program_id(0)
    ih = pl.program_id(1)

    @pl.when(jnp.logical_and(b == 0, ih == 0))
    def _init():
        acc_s[...] = jnp.zeros_like(acc_s)
        acc_c[...] = jnp.zeros_like(acc_c)

    se = None
    for c in range(_C):
        sc = s_ref[0, c]
        tc = t_ref[0, c]
        if _TAU != 1.0:
            sc = sc / _TAU
            tc = tc / _TAU
        esc = jnp.exp(sc)
        etc = jnp.exp(tc)
        wc = etc * (tc - sc)
        if se is None:
            se, te, we = esc, etc, wc
        else:
            se = se + esc
            te = te + etc
            we = we + wc

    kl = we / te + jnp.log(se) - jnp.log(te)

    gt = gt_ref[0, 0]
    ones = jnp.ones_like(kl)
    for c in range(1, _C):
        m = gt == c
        fk = _fold(jnp.where(m, kl, 0.0))
        fc = _fold(jnp.where(m, ones, 0.0))
        row = pl.ds(b * 128 + c * 8, 8)
        acc_s[row, :] += fk
        acc_c[row, :] += fc

    @pl.when(jnp.logical_and(b == n_b - 1, ih == n_hblk - 1))
    def _finish():
        sums3 = acc_s[...].reshape(64, 8, 128)
        cnts3 = acc_c[...].reshape(64, 8, 128)
        sums = jnp.sum(jnp.sum(sums3, axis=1), axis=1, keepdims=True)
        cnts = jnp.sum(jnp.sum(cnts3, axis=1), axis=1, keepdims=True)
        rid = jax.lax.broadcasted_iota(jnp.int32, sums.shape, 0)
        cid = jax.lax.bitwise_and(rid, 15)
        valid = jnp.logical_and(cid >= 1, cid <= _C - 1)
        valid = jnp.logical_and(valid, cnts > 0.0)
        per = jnp.where(valid, sums / (_C * jnp.maximum(cnts, 1.0)), 0.0)
        out_ref[...] = jnp.sum(per, axis=0, keepdims=True) * (_TAU ** 2) * _LOSS_WEIGHT


def kernel(preds_S, preds_T, gt_labels):
    B, C, H, W = preds_S.shape
    BLK_H = 128
    n_hblk = H // BLK_H

    gt = gt_labels.astype(jnp.int32)

    out = pl.pallas_call(
        functools.partial(_kl_loss_kernel, n_hblk=n_hblk, n_b=B),
        grid=(B, n_hblk),
        in_specs=[
            pl.BlockSpec((1, 1, BLK_H, W), lambda b, ih: (b, 0, ih, 0)),
            pl.BlockSpec((1, C, BLK_H, W), lambda b, ih: (b, 0, ih, 0)),
            pl.BlockSpec((1, C, BLK_H, W), lambda b, ih: (b, 0, ih, 0)),
        ],
        out_specs=pl.BlockSpec((1, 1), lambda b, ih: (0, 0)),
        out_shape=jax.ShapeDtypeStruct((1, 1), jnp.float32),
        scratch_shapes=[
            pltpu.VMEM((512, 128), jnp.float32),
            pltpu.VMEM((512, 128), jnp.float32),
        ],
        compiler_params=pltpu.CompilerParams(
            dimension_semantics=("arbitrary", "arbitrary"),
        ),
    )(gt, preds_S, preds_T)
    return out[0, 0]
